# baseline (speedup 1.0000x reference)
import functools

import jax
import jax.numpy as jnp
from jax import lax
from jax.experimental import pallas as pl
from jax.experimental.pallas import tpu as pltpu
from jax.experimental.pallas import tpu_sc as plsc


def kernel(idx, token_embedding_table, pos_encoding):
    B, T = idx.shape
    V, D = token_embedding_table.shape
    G = T // 2
    SEG_PER_SEQ = 2
    SEQ_PER_CHUNK = 2
    NSEG = SEG_PER_SEQ * SEQ_PER_CHUNK

    info = plsc.get_sparse_core_info()
    NC, NS = info.num_cores, info.num_subcores
    NW = NC * NS
    n_chunks = B // SEQ_PER_CHUNK
    chunks_per_w = n_chunks // NW
    segs_per_w = chunks_per_w * NSEG

    idx2 = idx.reshape(B * SEG_PER_SEQ, G)
    pe_chunk = jnp.tile(pos_encoding.reshape(SEG_PER_SEQ, G, D),
                        (SEQ_PER_CHUNK, 1, 1))

    mesh = plsc.VectorSubcoreMesh(core_axis_name="c", subcore_axis_name="s")

    @functools.partial(
        pl.kernel,
        mesh=mesh,
        out_type=jax.ShapeDtypeStruct((n_chunks, NSEG, G, D), jnp.float32),
        scratch_types=[
            pltpu.VMEM((segs_per_w, G), jnp.int32),
            pltpu.VMEM((2, NSEG, G, D), jnp.float32),
            pltpu.SemaphoreType.DMA,
            pltpu.SemaphoreType.DMA,
            pltpu.SemaphoreType.DMA,
            pltpu.SemaphoreType.DMA,
            pltpu.SemaphoreType.DMA,
            pltpu.SemaphoreType.DMA,
        ],
        compiler_params=pltpu.CompilerParams(use_tc_tiling_on_sc=False),
    )
    def run(idx_hbm, table_hbm, pe_hbm, out_hbm, idx_all,
            rows, g0, g1, o0, o1, p0, p1):
        wid = lax.axis_index("s") * NC + lax.axis_index("c")
        base_c = wid * chunks_per_w
        pltpu.sync_copy(idx_hbm.at[pl.ds(wid * segs_per_w, segs_per_w)], idx_all)
        gsem = (g0, g1)
        osem = (o0, o1)
        psem = (p0, p1)

        def fire_seed(s):
            pltpu.async_copy(pe_hbm, rows.at[s], psem[s])

        def wait_seed(s):
            pltpu.make_async_copy(pe_hbm, rows.at[s], psem[s]).wait()

        def fire_gathers(t, s):
            for j in range(NSEG):
                pltpu.async_copy(
                    table_hbm.at[idx_all.at[t * NSEG + j]], rows.at[s].at[j],
                    gsem[s], add=True,
                )

        def wait_gathers(s):
            pltpu.make_async_copy(out_hbm.at[0], rows.at[s], gsem[s]).wait()

        def wait_out(s):
            pltpu.make_async_copy(out_hbm.at[0], rows.at[s], osem[s]).wait()

        def fire_out(t, s):
            pltpu.async_copy(rows.at[s], out_hbm.at[base_c + t], osem[s])

        def body(t, s, first):
            o = 1 - s
            if not first:
                wait_out(o)
            fire_seed(o)
            wait_gathers(s)
            fire_out(t, s)
            wait_seed(o)
            fire_gathers(t + 1, o)

        fire_seed(0)
        wait_seed(0)
        fire_gathers(0, 0)
        body(0, 0, first=True)

        def pair(p, carry):
            t1 = 2 * p + 1
            body(t1, 1, first=False)
            body(t1 + 1, 0, first=False)
            return carry

        lax.fori_loop(0, (chunks_per_w - 2) // 2, pair, 0)

        wait_gathers(1)
        fire_out(chunks_per_w - 1, 1)
        wait_out(0)
        wait_out(1)

    out = run(idx2, token_embedding_table, pe_chunk)
    return out.reshape(B, T, D)


# speedup vs baseline: 1.1706x; 1.1706x over previous
import functools

import jax
import jax.numpy as jnp
from jax import lax
from jax.experimental import pallas as pl
from jax.experimental.pallas import tpu as pltpu
from jax.experimental.pallas import tpu_sc as plsc


def kernel(idx, token_embedding_table, pos_encoding):
    B, T = idx.shape
    V, D = token_embedding_table.shape
    G = T // 2
    SEG_PER_SEQ = 2
    SEQ_PER_CHUNK = 2
    NSEG = SEG_PER_SEQ * SEQ_PER_CHUNK
    VL = 16
    KD = D // VL

    info = plsc.get_sparse_core_info()
    NC, NS = info.num_cores, info.num_subcores
    NW = NC * NS
    n_chunks = B // SEQ_PER_CHUNK
    chunks_per_w = n_chunks // NW
    segs_per_w = chunks_per_w * NSEG

    idx2 = idx.reshape(B * SEG_PER_SEQ, G)
    pe_chunk = jnp.tile(pos_encoding.reshape(SEG_PER_SEQ, G, D),
                        (SEQ_PER_CHUNK, 1, 1))

    mesh = plsc.VectorSubcoreMesh(core_axis_name="c", subcore_axis_name="s")

    @functools.partial(
        pl.kernel,
        mesh=mesh,
        out_type=jax.ShapeDtypeStruct((n_chunks, NSEG, G, D), jnp.float32),
        scratch_types=[
            pltpu.VMEM((segs_per_w, G), jnp.int32),
            pltpu.VMEM((NSEG, G, D), jnp.float32),
            pltpu.VMEM((2, NSEG, G, D), jnp.float32),
            pltpu.SemaphoreType.DMA,
            pltpu.SemaphoreType.DMA,
            pltpu.SemaphoreType.DMA,
            pltpu.SemaphoreType.DMA,
        ],
        compiler_params=pltpu.CompilerParams(use_tc_tiling_on_sc=False),
    )
    def run(idx_hbm, table_hbm, pe_hbm, out_hbm, idx_all, pe_v,
            rows, g0, g1, o0, o1):
        wid = lax.axis_index("s") * NC + lax.axis_index("c")
        base_c = wid * chunks_per_w
        pltpu.sync_copy(pe_hbm, pe_v)
        pltpu.sync_copy(idx_hbm.at[pl.ds(wid * segs_per_w, segs_per_w)], idx_all)
        gsem = (g0, g1)
        osem = (o0, o1)

        def fire_gathers(t, s):
            for j in range(NSEG):
                pltpu.async_copy(
                    table_hbm.at[idx_all.at[t * NSEG + j]], rows.at[s].at[j],
                    gsem[s],
                )

        def wait_gathers(s):
            pltpu.make_async_copy(out_hbm.at[0], rows.at[s], gsem[s]).wait()

        def wait_out(s):
            pltpu.make_async_copy(out_hbm.at[0], rows.at[s], osem[s]).wait()

        def fire_out(t, s):
            pltpu.async_copy(rows.at[s], out_hbm.at[base_c + t], osem[s])

        def add_pe(s):
            def seg(j):
                def vbody(p, carry):
                    for k in range(KD):
                        sl = pl.ds(k * VL, VL)
                        rows[s, j, p, sl] = rows[s, j, p, sl] + pe_v[j, p, sl]
                    return carry
                lax.fori_loop(0, G, vbody, 0)
            for j in range(NSEG):
                seg(j)

        def body(t, s, first):
            o = 1 - s
            wait_gathers(s)
            add_pe(s)
            fire_out(t, s)
            if not first:
                wait_out(o)
            fire_gathers(t + 1, o)

        fire_gathers(0, 0)
        body(0, 0, first=True)

        def pair(p, carry):
            t1 = 2 * p + 1
            body(t1, 1, first=False)
            body(t1 + 1, 0, first=False)
            return carry

        lax.fori_loop(0, (chunks_per_w - 2) // 2, pair, 0)

        wait_gathers(1)
        add_pe(1)
        fire_out(chunks_per_w - 1, 1)
        wait_out(0)
        wait_out(1)

    out = run(idx2, token_embedding_table, pe_chunk)
    return out.reshape(B, T, D)


# speedup vs baseline: 1.2396x; 1.0590x over previous
import functools

import jax
import jax.numpy as jnp
from jax import lax
from jax.experimental import pallas as pl
from jax.experimental.pallas import tpu as pltpu
from jax.experimental.pallas import tpu_sc as plsc


def kernel(idx, token_embedding_table, pos_encoding):
    B, T = idx.shape
    V, D = token_embedding_table.shape
    G = T // 2
    SEG_PER_SEQ = 2
    SEQ_PER_CHUNK = 2
    NSEG = SEG_PER_SEQ * SEQ_PER_CHUNK
    VL = 16
    KD = D // VL

    info = plsc.get_sparse_core_info()
    NC, NS = info.num_cores, info.num_subcores
    NW = NC * NS
    n_chunks = B // SEQ_PER_CHUNK
    chunks_per_w = n_chunks // NW
    segs_per_w = chunks_per_w * NSEG

    idx2 = idx.reshape(B * SEG_PER_SEQ, G)
    pe_chunk = jnp.tile(pos_encoding.reshape(SEG_PER_SEQ, G, D),
                        (SEQ_PER_CHUNK, 1, 1))

    mesh = plsc.VectorSubcoreMesh(core_axis_name="c", subcore_axis_name="s")

    @functools.partial(
        pl.kernel,
        mesh=mesh,
        out_type=jax.ShapeDtypeStruct((n_chunks, NSEG, G, D), jnp.float32),
        scratch_types=[
            pltpu.VMEM((segs_per_w, G), jnp.int32),
            pltpu.VMEM((NSEG, G, D), jnp.float32),
            pltpu.VMEM((2, NSEG, G, D), jnp.float32),
            pltpu.SemaphoreType.DMA,
            pltpu.SemaphoreType.DMA,
            pltpu.SemaphoreType.DMA,
            pltpu.SemaphoreType.DMA,
        ],
        compiler_params=pltpu.CompilerParams(use_tc_tiling_on_sc=False),
    )
    def run(idx_hbm, table_hbm, pe_hbm, out_hbm, idx_all, pe_v,
            rows, g0, g1, o0, o1):
        wid = lax.axis_index("s") * NC + lax.axis_index("c")
        base_c = wid * chunks_per_w
        pltpu.sync_copy(pe_hbm, pe_v)
        pltpu.sync_copy(idx_hbm.at[pl.ds(wid * segs_per_w, segs_per_w)], idx_all)
        gsem = (g0, g1)
        osem = (o0, o1)

        def fire_gathers(t, s):
            for j in range(NSEG):
                pltpu.async_copy(
                    table_hbm.at[idx_all.at[t * NSEG + j]], rows.at[s].at[j],
                    gsem[s],
                )

        def wait_gathers(s):
            pltpu.make_async_copy(out_hbm.at[0], rows.at[s], gsem[s]).wait()

        def wait_out(s):
            pltpu.make_async_copy(out_hbm.at[0], rows.at[s], osem[s]).wait()

        def fire_out(t, s):
            pltpu.async_copy(rows.at[s], out_hbm.at[base_c + t], osem[s])

        def add_pe(s):
            def seg(j):
                def vbody(p, carry):
                    for k in range(KD):
                        sl = pl.ds(k * VL, VL)
                        rows[s, j, p, sl] = rows[s, j, p, sl] + pe_v[j, p, sl]
                    return carry
                lax.fori_loop(0, G, vbody, 0)
            for j in range(NSEG):
                seg(j)

        def body(t, s, first):
            o = 1 - s
            wait_gathers(s)
            if not first:
                wait_out(o)
            fire_gathers(t + 1, o)
            add_pe(s)
            fire_out(t, s)

        fire_gathers(0, 0)
        body(0, 0, first=True)

        def pair(p, carry):
            t1 = 2 * p + 1
            body(t1, 1, first=False)
            body(t1 + 1, 0, first=False)
            return carry

        lax.fori_loop(0, (chunks_per_w - 2) // 2, pair, 0)

        wait_gathers(1)
        add_pe(1)
        fire_out(chunks_per_w - 1, 1)
        wait_out(0)
        wait_out(1)

    out = run(idx2, token_embedding_table, pe_chunk)
    return out.reshape(B, T, D)


# speedup vs baseline: 1.2408x; 1.0010x over previous
import functools

import jax
import jax.numpy as jnp
from jax import lax
from jax.experimental import pallas as pl
from jax.experimental.pallas import tpu as pltpu
from jax.experimental.pallas import tpu_sc as plsc


def kernel(idx, token_embedding_table, pos_encoding):
    B, T = idx.shape
    V, D = token_embedding_table.shape
    G = T // 2
    SEG_PER_SEQ = 2
    SEQ_PER_CHUNK = 2
    NSEG = SEG_PER_SEQ * SEQ_PER_CHUNK
    NSLOT = 3
    VL = 16
    KD = D // VL

    info = plsc.get_sparse_core_info()
    NC, NS = info.num_cores, info.num_subcores
    NW = NC * NS
    n_chunks = B // SEQ_PER_CHUNK
    chunks_per_w = n_chunks // NW
    segs_per_w = chunks_per_w * NSEG

    idx2 = idx.reshape(B * SEG_PER_SEQ, G)
    pe_seq = pos_encoding.reshape(SEG_PER_SEQ, G, D)

    mesh = plsc.VectorSubcoreMesh(core_axis_name="c", subcore_axis_name="s")

    @functools.partial(
        pl.kernel,
        mesh=mesh,
        out_type=jax.ShapeDtypeStruct((n_chunks, NSEG, G, D), jnp.float32),
        scratch_types=[
            pltpu.VMEM((segs_per_w, G), jnp.int32),
            pltpu.VMEM((SEG_PER_SEQ, G, D), jnp.float32),
            pltpu.VMEM((NSLOT, NSEG, G, D), jnp.float32),
            pltpu.SemaphoreType.DMA,
            pltpu.SemaphoreType.DMA,
            pltpu.SemaphoreType.DMA,
            pltpu.SemaphoreType.DMA,
            pltpu.SemaphoreType.DMA,
            pltpu.SemaphoreType.DMA,
        ],
        compiler_params=pltpu.CompilerParams(use_tc_tiling_on_sc=False),
    )
    def run(idx_hbm, table_hbm, pe_hbm, out_hbm, idx_all, pe_v,
            rows, g0, g1, g2, o0, o1, o2):
        wid = lax.axis_index("s") * NC + lax.axis_index("c")
        base_c = wid * chunks_per_w
        pltpu.sync_copy(pe_hbm, pe_v)
        pltpu.sync_copy(idx_hbm.at[pl.ds(wid * segs_per_w, segs_per_w)], idx_all)
        gsem = (g0, g1, g2)
        osem = (o0, o1, o2)

        def fire_gathers(t, s):
            for j in range(NSEG):
                pltpu.async_copy(
                    table_hbm.at[idx_all.at[t * NSEG + j]], rows.at[s].at[j],
                    gsem[s],
                )

        def wait_gathers(s):
            pltpu.make_async_copy(out_hbm.at[0], rows.at[s], gsem[s]).wait()

        def wait_out(s):
            pltpu.make_async_copy(out_hbm.at[0], rows.at[s], osem[s]).wait()

        def fire_out(t, s):
            pltpu.async_copy(rows.at[s], out_hbm.at[base_c + t], osem[s])

        def add_pe(s):
            for j in range(NSEG):
                jp = j % SEG_PER_SEQ

                def vbody(p, carry, j=j, jp=jp):
                    for k in range(KD):
                        sl = pl.ds(k * VL, VL)
                        rows[s, j, p, sl] = rows[s, j, p, sl] + pe_v[jp, p, sl]
                    return carry
                lax.fori_loop(0, G, vbody, 0)

        def body(t, s, first):
            s2 = (s + 2) % NSLOT
            wait_gathers(s)
            if not first:
                wait_out(s2)
            fire_gathers(t + 2, s2)
            add_pe(s)
            fire_out(t, s)

        fire_gathers(0, 0)
        fire_gathers(1, 1)
        body(0, 0, first=True)

        n_triples = (chunks_per_w - 4) // NSLOT

        def triple(p, carry):
            t1 = NSLOT * p + 1
            body(t1, 1 % NSLOT, first=False)
            body(t1 + 1, 2 % NSLOT, first=False)
            body(t1 + 2, 0, first=False)
            return carry

        lax.fori_loop(0, n_triples, triple, 0)

        n = chunks_per_w
        body(n - 3, (n - 3) % NSLOT, first=False)
        wait_gathers((n - 2) % NSLOT)
        add_pe((n - 2) % NSLOT)
        fire_out(n - 2, (n - 2) % NSLOT)
        wait_gathers((n - 1) % NSLOT)
        add_pe((n - 1) % NSLOT)
        fire_out(n - 1, (n - 1) % NSLOT)
        for s in range(NSLOT):
            wait_out(s)

    out = run(idx2, token_embedding_table, pe_seq)
    return out.reshape(B, T, D)
